# baseline (speedup 1.0000x reference)
import math

import jax
import jax.numpy as jnp
from jax import lax
from jax.experimental import pallas as pl
from jax.experimental.pallas import tpu as pltpu
from jax.experimental.pallas import tpu_sc as plsc
import numpy as np

_VOCAB = 1000000
_D = 128
_MAX_LEN = 200
_NUM_LAYERS = 6
_B = 4096
_L = 200

_NC = 2
_NS = 16
_NW = _NC * _NS

_CHUNK = _B // _NW
_PW = 2
_BW = _CHUNK // 2
_SCALE = math.sqrt(float(_D))
_LANES = 16
_VPR = _D // _LANES
_UNROLL = 8


def _sincos_table(max_len, d_model):
    pe = np.zeros((max_len, d_model), dtype=np.float32)
    pos = np.arange(max_len, dtype=np.float64)[:, None]
    i = np.arange(0, d_model, 2, dtype=np.float64)
    pe[:, 0::2] = np.sin(pos / np.power(10000.0, 2.0 * i / d_model)).astype(np.float32)
    pe[:, 1::2] = np.cos(pos / np.power(10000.0, 2.0 * (i + 1.0) / d_model)).astype(np.float32)
    return pe


_PE = _sincos_table(_MAX_LEN, _D)
_TE = _sincos_table(_NUM_LAYERS, _D)


_NG = 3
_NO = 2
_STEP = 6
_MAIN = (_L // _STEP) * _STEP


_NCHUNK = _L * _CHUNK // (_PW * _BW)


def _body(table_hbm, idx_hbm, pe_hbm, out_hbm,
          idx0, idx1, idx2, g0, g1, g2, o0, o1, pe_v,
          gs0, gs1, gs2, ss0, ss1, is0, is1, is2):
    idxv, gbuf, obuf = [idx0, idx1, idx2], [g0, g1, g2], [o0, o1]
    gsem, ssem, isem = [gs0, gs1, gs2], [ss0, ss1], [is0, is1, is2]

    wid = lax.axis_index("s") * _NC + lax.axis_index("c")
    b0 = wid * _CHUNK
    pltpu.sync_copy(pe_hbm, pe_v)

    def locate(c):
        pos = (c // 2) * 2
        bb = b0 + (c % 2) * _BW
        return pos, bb

    def idx_copies(c, g):
        pos, bb = locate(c)
        return [
            pltpu.make_async_copy(
                idx_hbm.at[pl.ds((pos + p) * _B + bb, _BW)],
                idxv[g].at[pl.ds(p * _BW, _BW)], isem[g])
            for p in range(_PW)
        ]

    def start_gather(c, g, sync):
        for cp in idx_copies(c, g):
            if sync:
                cp.start()
            cp.wait()
        pltpu.make_async_copy(table_hbm.at[idxv[g]], gbuf[g], gsem[g]).start()

    for g in range(_NG):
        start_gather(g, g, sync=True)

    def chunk(c, g, o, refill):
        pos, bb = locate(c)
        pltpu.make_async_copy(table_hbm.at[idxv[g]], gbuf[g], gsem[g]).wait()

        if refill:
            @pl.when(c + _NG < _NCHUNK)
            def _():
                for cp in idx_copies(c + _NG, g):
                    cp.start()

        @pl.when(c >= _NO)
        def _():
            poso, bbo = locate(c - _NO)
            pltpu.make_async_copy(
                obuf[o],
                out_hbm.at[pl.ds(bbo, _BW), pl.ds(poso, _PW), :],
                ssem[o]).wait()

        vp = [[pe_v[pos + p, pl.ds(j * _LANES, _LANES)] for j in range(_VPR)]
              for p in range(_PW)]

        def blk(t, carry2):
            for r in range(_UNROLL):
                i = t * _UNROLL + r
                for p in range(_PW):
                    for j in range(_VPR):
                        sl = pl.ds(j * _LANES, _LANES)
                        obuf[o][i, p, sl] = gbuf[g][p * _BW + i, sl] * _SCALE + vp[p][j]
            return carry2

        lax.fori_loop(0, _BW // _UNROLL, blk, 0)

        pltpu.make_async_copy(
            obuf[o], out_hbm.at[pl.ds(bb, _BW), pl.ds(pos, _PW), :],
            ssem[o]).start()

        if refill:
            @pl.when(c + _NG < _NCHUNK)
            def _():
                start_gather(c + _NG, g, sync=False)

    def outer(k, carry):
        for u in range(_STEP):
            chunk(k * _STEP + u, u % _NG, u % _NO, refill=True)
        return carry

    lax.fori_loop(0, _MAIN // _STEP, outer, 0)

    for c in range(_MAIN, _NCHUNK):
        chunk(c, c % _NG, c % _NO, refill=False)

    for c in range(_NCHUNK - _NO, _NCHUNK):
        pos, bb = locate(c)
        pltpu.make_async_copy(
            obuf[c % _NO], out_hbm.at[pl.ds(bb, _BW), pl.ds(pos, _PW), :],
            ssem[c % _NO]).wait()


def kernel(embedding, layer_idx, embed_weight):
    pe = jnp.asarray(_PE)
    te_row = jnp.take(jnp.asarray(_TE), layer_idx, axis=0)
    pe_c = pe + te_row[None, :]

    idx_t = embedding.astype(jnp.int32).T.reshape(_L * _B)

    mesh = plsc.VectorSubcoreMesh(core_axis_name="c", subcore_axis_name="s")
    out = pl.kernel(
        _body,
        out_type=jax.ShapeDtypeStruct((_B, _L, _D), jnp.float32),
        mesh=mesh,
        scratch_types=(
            [pltpu.VMEM((_PW * _BW,), jnp.int32)] * _NG
            + [pltpu.VMEM((_PW * _BW, _D), jnp.float32)] * _NG
            + [pltpu.VMEM((_BW, _PW, _D), jnp.float32)] * _NO
            + [pltpu.VMEM((_MAX_LEN, _D), jnp.float32)]
            + [pltpu.SemaphoreType.DMA] * (_NG + _NO + _NG)
        ),
    )(embed_weight, idx_t, pe_c)
    return out


# speedup vs baseline: 3.8371x; 3.8371x over previous
import math

import jax
import jax.numpy as jnp
from jax import lax
from jax.experimental import pallas as pl
from jax.experimental.pallas import tpu as pltpu
from jax.experimental.pallas import tpu_sc as plsc
import numpy as np

_VOCAB = 1000000
_D = 128
_MAX_LEN = 200
_NUM_LAYERS = 6
_B = 4096
_L = 200

_NC = 2
_NS = 16
_NW = _NC * _NS

_CHUNK = _B // _NW
_SCALE = math.sqrt(float(_D))
_LANES = 16
_VPR = _D // _LANES
_UNROLL = 8


def _sincos_table(max_len, d_model):
    pe = np.zeros((max_len, d_model), dtype=np.float32)
    pos = np.arange(max_len, dtype=np.float64)[:, None]
    i = np.arange(0, d_model, 2, dtype=np.float64)
    pe[:, 0::2] = np.sin(pos / np.power(10000.0, 2.0 * i / d_model)).astype(np.float32)
    pe[:, 1::2] = np.cos(pos / np.power(10000.0, 2.0 * (i + 1.0) / d_model)).astype(np.float32)
    return pe


_PE = _sincos_table(_MAX_LEN, _D)
_TE = _sincos_table(_NUM_LAYERS, _D)


_NB = 2


def _body(table_hbm, idx_hbm, pe_hbm, out_hbm,
          idx_all, g0, g1, o0, o1, pe_v, gs0, gs1, ss0, ss1):
    gbuf, obuf = [g0, g1], [o0, o1]
    gsem, ssem = [gs0, gs1], [ss0, ss1]

    wid = lax.axis_index("s") * _NC + lax.axis_index("c")
    b0 = wid * _CHUNK
    pltpu.sync_copy(pe_hbm, pe_v)
    pltpu.sync_copy(idx_hbm.at[:, pl.ds(b0, _CHUNK)], idx_all)

    for g in range(_NB):
        pltpu.make_async_copy(table_hbm.at[idx_all.at[g]], gbuf[g],
                              gsem[g]).start()

    def chunk(c, g, refill):
        pltpu.make_async_copy(table_hbm.at[idx_all.at[c]], gbuf[g],
                              gsem[g]).wait()

        @pl.when(c >= _NB)
        def _():
            pltpu.make_async_copy(
                obuf[g],
                out_hbm.at[pl.ds(b0, _CHUNK), pl.ds(c - _NB, 1), :],
                ssem[g]).wait()

        vp = [pe_v[c, pl.ds(j * _LANES, _LANES)] for j in range(_VPR)]

        def blk(t, carry2):
            for r in range(_UNROLL):
                i = t * _UNROLL + r
                for j in range(_VPR):
                    sl = pl.ds(j * _LANES, _LANES)
                    obuf[g][i, 0, sl] = gbuf[g][i, sl] * _SCALE + vp[j]
            return carry2

        lax.fori_loop(0, _CHUNK // _UNROLL, blk, 0)

        pltpu.make_async_copy(
            obuf[g], out_hbm.at[pl.ds(b0, _CHUNK), pl.ds(c, 1), :],
            ssem[g]).start()

        if refill:
            @pl.when(c + _NB < _L)
            def _():
                pltpu.make_async_copy(table_hbm.at[idx_all.at[c + _NB]],
                                      gbuf[g], gsem[g]).start()

    def outer(k, carry):
        for u in range(_NB):
            chunk(k * _NB + u, u, refill=True)
        return carry

    lax.fori_loop(0, _L // _NB, outer, 0)

    for c in range(_L - _NB, _L):
        pltpu.make_async_copy(
            obuf[c % _NB], out_hbm.at[pl.ds(b0, _CHUNK), pl.ds(c, 1), :],
            ssem[c % _NB]).wait()


def kernel(embedding, layer_idx, embed_weight):
    pe = jnp.asarray(_PE)
    te_row = jnp.take(jnp.asarray(_TE), layer_idx, axis=0)
    pe_c = pe + te_row[None, :]

    idx_t = embedding.astype(jnp.int32).T

    mesh = plsc.VectorSubcoreMesh(core_axis_name="c", subcore_axis_name="s")
    out = pl.kernel(
        _body,
        out_type=jax.ShapeDtypeStruct((_B, _L, _D), jnp.float32),
        mesh=mesh,
        scratch_types=(
            [pltpu.VMEM((_L, _CHUNK), jnp.int32)]
            + [pltpu.VMEM((_CHUNK, _D), jnp.float32)] * _NB
            + [pltpu.VMEM((_CHUNK, 1, _D), jnp.float32)] * _NB
            + [pltpu.VMEM((_MAX_LEN, _D), jnp.float32)]
            + [pltpu.SemaphoreType.DMA] * (2 * _NB)
        ),
    )(embed_weight, idx_t, pe_c)
    return out
